# baseline (speedup 1.0000x reference)
import functools

import jax
import jax.numpy as jnp
from jax.experimental import pallas as pl
from jax.experimental.pallas import tpu as pltpu

_TB = 64


def _gemp_body(y_ref, o_ref, *, p, eps, inv_n):
    v = y_ref[...]
    s = jnp.sum(v * v * v, axis=0)
    o_ref[...] = ((s * inv_n + eps) ** (1.0 / p)).astype(o_ref.dtype)


def kernel(x):
    p, eps = 3.0, 1e-12
    B, C, H, W = x.shape
    n = H * W
    y = x.transpose(2, 3, 0, 1).reshape(n, B, C)

    body = functools.partial(_gemp_body, p=p, eps=eps, inv_n=1.0 / n)
    return pl.pallas_call(
        body,
        out_shape=jax.ShapeDtypeStruct((B, C), x.dtype),
        grid=(B // _TB,),
        in_specs=[pl.BlockSpec((n, _TB, C), lambda b: (0, b, 0))],
        out_specs=pl.BlockSpec((_TB, C), lambda b: (b, 0)),
        compiler_params=pltpu.CompilerParams(
            dimension_semantics=("parallel",)),
    )(y)


# speedup vs baseline: 1.0759x; 1.0759x over previous
import functools

import jax
import jax.numpy as jnp
from jax.experimental import pallas as pl
from jax.experimental.pallas import tpu as pltpu

_TB = 32


def _gemp_body(y_ref, o_ref, *, p, eps, inv_n):
    v = y_ref[...]
    s = jnp.sum(v * v * v, axis=0)
    o_ref[...] = ((s * inv_n + eps) ** (1.0 / p)).astype(o_ref.dtype)


def kernel(x):
    p, eps = 3.0, 1e-12
    B, C, H, W = x.shape
    n = H * W
    y = x.transpose(2, 3, 0, 1).reshape(n, B, C)

    body = functools.partial(_gemp_body, p=p, eps=eps, inv_n=1.0 / n)
    return pl.pallas_call(
        body,
        out_shape=jax.ShapeDtypeStruct((B, C), x.dtype),
        grid=(B // _TB,),
        in_specs=[pl.BlockSpec((n, _TB, C), lambda b: (0, b, 0))],
        out_specs=pl.BlockSpec((_TB, C), lambda b: (b, 0)),
        compiler_params=pltpu.CompilerParams(
            dimension_semantics=("parallel",)),
    )(y)
